# baseline (speedup 1.0000x reference)
import functools

import jax
import jax.numpy as jnp
from jax import lax
from jax.experimental import pallas as pl
from jax.experimental.pallas import tpu as pltpu
from jax.experimental.pallas import tpu_sc as plsc

_B = 16384
_C = 1000
_NC = 2
_NS = 16
_NW = _NC * _NS
_RPW = _B // _NW
_CHUNK = 128
_NCHUNK = _RPW // _CHUNK
_L = 16


def _body(x_hbm, out_hbm, x_v, buf):
    wid = lax.axis_index("s") * _NC + lax.axis_index("c")
    base_row = wid * _RPW

    pltpu.sync_copy(x_hbm.at[pl.ds(base_row, _RPW)], x_v)

    zvec = jnp.zeros((_L,), jnp.float32)
    ones = jnp.ones((_L,), jnp.float32)
    iota = lax.iota(jnp.int32, _L)

    @pl.loop(0, _CHUNK * _C // (8 * _L))
    def _zero(i):
        for u in range(8):
            buf[pl.ds(i * (8 * _L) + u * _L, _L)] = zvec

    def flat_idx(c, j):
        rows = j * _L + iota
        xv = x_v[pl.ds(c * _CHUNK + j * _L, _L)]
        return rows * _C + xv

    for c in range(_NCHUNK):
        for j in range(_CHUNK // _L):
            plsc.store_scatter(buf, [flat_idx(c, j)], ones)
        pltpu.sync_copy(
            buf, out_hbm.at[pl.ds((base_row + c * _CHUNK) * _C, _CHUNK * _C)]
        )
        if c + 1 < _NCHUNK:
            for j in range(_CHUNK // _L):
                plsc.store_scatter(buf, [flat_idx(c, j)], zvec)


_onehot_sc = pl.kernel(
    _body,
    out_type=jax.ShapeDtypeStruct((_B * _C,), jnp.float32),
    mesh=plsc.VectorSubcoreMesh(core_axis_name="c", subcore_axis_name="s"),
    scratch_types=[
        pltpu.VMEM((_RPW,), jnp.int32),
        pltpu.VMEM((_CHUNK * _C,), jnp.float32),
    ],
    compiler_params=pltpu.CompilerParams(needs_layout_passes=False),
)


@jax.jit
def kernel(x):
    x = jnp.squeeze(x).astype(jnp.int32)
    return _onehot_sc(x).reshape(_B, _C)


# speedup vs baseline: 2.0441x; 2.0441x over previous
import jax
import jax.numpy as jnp
from jax import lax
from jax.experimental import pallas as pl
from jax.experimental.pallas import tpu as pltpu

_B = 16384
_C = 1000
_ROWS = 1024
_GRID = _B // _ROWS


def _tc_body(x_ref, o_ref):
    x = x_ref[...]
    cls = lax.broadcasted_iota(jnp.int32, (_ROWS, _C), 1)
    o_ref[...] = (x[:, None] == cls).astype(jnp.float32)


_onehot_tc = pl.pallas_call(
    _tc_body,
    grid=(_GRID,),
    in_specs=[pl.BlockSpec((_ROWS,), lambda i: (i,))],
    out_specs=pl.BlockSpec((_ROWS, _C), lambda i: (i, 0)),
    out_shape=jax.ShapeDtypeStruct((_B, _C), jnp.float32),
)


@jax.jit
def kernel(x):
    x = jnp.squeeze(x).astype(jnp.int32)
    return _onehot_tc(x)
